# baseline (speedup 1.0000x reference)
import functools

import jax
import jax.numpy as jnp
from jax import lax
from jax.experimental import pallas as pl
from jax.experimental.pallas import tpu as pltpu
from jax.experimental.pallas import tpu_sc as plsc

NC = 2
NS = 16
NW = NC * NS
NBUF = 4


def _sc_gather_sum(feat_q, neigh_idx, node_idx, *, B, K, D2, C):
    rows_per_w = B // NW
    nchunk = rows_per_w // C
    assert nchunk % NBUF == 0
    mesh = plsc.VectorSubcoreMesh(
        core_axis_name="c", subcore_axis_name="s", num_cores=NC, num_subcores=NS
    )

    @functools.partial(
        pl.kernel,
        out_type=(
            jax.ShapeDtypeStruct((B, D2), jnp.int32),
            jax.ShapeDtypeStruct((B, D2), jnp.int32),
        ),
        mesh=mesh,
        compiler_params=pltpu.CompilerParams(needs_layout_passes=False),
        scratch_types=[
            pltpu.VMEM((nchunk, C * K), jnp.int32),
            pltpu.VMEM((nchunk, C), jnp.int32),
            pltpu.VMEM((NBUF, C * K, D2), jnp.int32),
            pltpu.VMEM((NBUF, C, D2), jnp.int32),
            pltpu.VMEM((NBUF, C, D2), jnp.int32),
            pltpu.SemaphoreType.DMA((NBUF,)),
            pltpu.SemaphoreType.DMA((NBUF,)),
            pltpu.SemaphoreType.DMA((NBUF,)),
            pltpu.SemaphoreType.DMA((NBUF,)),
        ],
    )
    def k(feat_hbm, nidx_hbm, sidx_hbm, self_hbm, agg_hbm,
          nidx_v, sidx_v, gbuf, sbuf, obuf, gsem, ssem, osem, psem):
        wid = lax.axis_index("s") * NC + lax.axis_index("c")
        pltpu.sync_copy(nidx_hbm.at[wid], nidx_v)
        pltpu.sync_copy(sidx_hbm.at[wid], sidx_v)

        def issue(i, slot):
            pltpu.async_copy(feat_hbm.at[nidx_v.at[i]], gbuf.at[slot],
                             gsem.at[slot])
            pltpu.async_copy(feat_hbm.at[sidx_v.at[i]], sbuf.at[slot],
                             ssem.at[slot])

        def g_wait(i, slot):
            pltpu.make_async_copy(feat_hbm.at[nidx_v.at[i]], gbuf.at[slot],
                                  gsem.at[slot]).wait()
            pltpu.make_async_copy(feat_hbm.at[sidx_v.at[i]], sbuf.at[slot],
                                  ssem.at[slot]).wait()

        def agg_out_wait(i, slot):
            row_base = (wid * nchunk + i) * C
            pltpu.make_async_copy(obuf.at[slot], agg_hbm.at[pl.ds(row_base, C)],
                                  osem.at[slot]).wait()

        def self_out_wait(i, slot):
            row_base = (wid * nchunk + i) * C
            pltpu.make_async_copy(sbuf.at[slot], self_hbm.at[pl.ds(row_base, C)],
                                  psem.at[slot]).wait()

        for j in range(NBUF - 1):
            issue(j, j)

        def quad_body(p, carry):
            for b in range(NBUF):
                i = NBUF * p + b
                nxt = i + (NBUF - 1)
                nslot = (b + NBUF - 1) % NBUF

                @pl.when(nxt < nchunk)
                def _():
                    @pl.when(i >= 1)
                    def _():
                        self_out_wait(i - 1, nslot)

                    issue(nxt, nslot)

                g_wait(i, b)

                @pl.when(i >= NBUF)
                def _():
                    agg_out_wait(i - NBUF, b)

                for r in range(C):
                    for c in range(D2 // 16):
                        sl = pl.ds(c * 16, 16)

                        def tsum(lo, hi):
                            if hi - lo == 1:
                                return plsc.bitcast(gbuf[b, r * K + lo, sl],
                                                    jnp.bfloat16)
                            mid = (lo + hi) // 2
                            return tsum(lo, mid) + tsum(mid, hi)

                        obuf[b, r, sl] = plsc.bitcast(tsum(0, K), jnp.int32)

                row_base = (wid * nchunk + i) * C
                pltpu.async_copy(obuf.at[b], agg_hbm.at[pl.ds(row_base, C)],
                                 osem.at[b])
                pltpu.async_copy(sbuf.at[b], self_hbm.at[pl.ds(row_base, C)],
                                 psem.at[b])
            return carry

        lax.fori_loop(0, nchunk // NBUF, quad_body, 0)
        for j in range(NBUF):
            agg_out_wait(nchunk - NBUF + j, j)
            self_out_wait(nchunk - NBUF + j, j)

    return k(feat_q, neigh_idx, node_idx)


def _tc_matmul_relu(self_feats, agg_sum, w_top, w_bot, inv_k):
    B, D = self_feats.shape
    E = w_top.shape[1]
    BLK = 512

    def mm(s_ref, a_ref, wt_ref, wb_ref, o_ref):
        acc = jnp.dot(s_ref[...], wt_ref[...], preferred_element_type=jnp.float32)
        acc = acc + jnp.dot(a_ref[...], wb_ref[...],
                            preferred_element_type=jnp.float32) * inv_k
        o_ref[...] = jnp.maximum(acc, 0.0)

    return pl.pallas_call(
        mm,
        grid=(B // BLK,),
        in_specs=[
            pl.BlockSpec((BLK, D), lambda i: (i, 0)),
            pl.BlockSpec((BLK, D), lambda i: (i, 0)),
            pl.BlockSpec((D, E), lambda i: (0, 0)),
            pl.BlockSpec((D, E), lambda i: (0, 0)),
        ],
        out_specs=pl.BlockSpec((BLK, E), lambda i: (i, 0)),
        out_shape=jax.ShapeDtypeStruct((B, E), jnp.float32),
    )(self_feats, agg_sum, w_top, w_bot)


def kernel(nodes, to_neighs, features, weight):
    B, K = to_neighs.shape
    N, D = features.shape
    D2 = D // 2
    C = 2
    rows_per_w = B // NW
    nchunk = rows_per_w // C

    feat_bf = features.astype(jnp.bfloat16)
    feat_q = lax.bitcast_convert_type(feat_bf.reshape(N, D2, 2), jnp.int32)
    w_bf = weight.astype(jnp.bfloat16)
    neigh_idx = to_neighs.astype(jnp.int32).reshape(NW, nchunk, C * K)
    node_idx = nodes.astype(jnp.int32).reshape(NW, nchunk, C)

    self_q, agg_q = _sc_gather_sum(
        feat_q, neigh_idx, node_idx, B=B, K=K, D2=D2, C=C
    )
    self_feats = lax.bitcast_convert_type(self_q, jnp.bfloat16).reshape(B, D)
    agg_sum = lax.bitcast_convert_type(agg_q, jnp.bfloat16).reshape(B, D)
    return _tc_matmul_relu(self_feats, agg_sum, w_bf[:D], w_bf[D:], 1.0 / K)


# speedup vs baseline: 1.0596x; 1.0596x over previous
import functools

import jax
import jax.numpy as jnp
from jax import lax
from jax.experimental import pallas as pl
from jax.experimental.pallas import tpu as pltpu
from jax.experimental.pallas import tpu_sc as plsc

NC = 2
NS = 16
NW = NC * NS
NBUF = 4


def _sc_gather_sum(feat_q, neigh_idx, node_idx, *, B, K, D2, C):
    rows_per_w = B // NW
    nchunk = rows_per_w // C
    nself = node_idx.shape[1]
    sper = rows_per_w // nself
    assert nchunk % NBUF == 0
    mesh = plsc.VectorSubcoreMesh(
        core_axis_name="c", subcore_axis_name="s", num_cores=NC, num_subcores=NS
    )

    @functools.partial(
        pl.kernel,
        out_type=(
            jax.ShapeDtypeStruct((B, D2), jnp.int32),
            jax.ShapeDtypeStruct((B, D2), jnp.int32),
        ),
        mesh=mesh,
        compiler_params=pltpu.CompilerParams(needs_layout_passes=False),
        scratch_types=[
            pltpu.VMEM((nchunk, C * K), jnp.int32),
            pltpu.VMEM((nself, sper), jnp.int32),
            pltpu.VMEM((NBUF, C * K, D2), jnp.int32),
            pltpu.VMEM((rows_per_w, D2), jnp.int32),
            pltpu.VMEM((rows_per_w, D2), jnp.int32),
            pltpu.SemaphoreType.DMA((NBUF,)),
            pltpu.SemaphoreType.DMA((2,)),
        ],
    )
    def k(feat_hbm, nidx_hbm, sidx_hbm, self_hbm, agg_hbm,
          nidx_v, sidx_v, gbuf, sbuf, obuf, gsem, ssem):
        wid = lax.axis_index("s") * NC + lax.axis_index("c")
        pltpu.sync_copy(nidx_hbm.at[wid], nidx_v)
        pltpu.sync_copy(sidx_hbm.at[wid], sidx_v)

        def issue(i, slot):
            pltpu.async_copy(feat_hbm.at[nidx_v.at[i]], gbuf.at[slot],
                             gsem.at[slot])

        def g_wait(i, slot):
            pltpu.make_async_copy(feat_hbm.at[nidx_v.at[i]], gbuf.at[slot],
                                  gsem.at[slot]).wait()

        for h in range(nself):
            pltpu.async_copy(feat_hbm.at[sidx_v.at[h]],
                             sbuf.at[pl.ds(h * sper, sper)], ssem.at[h])
        for j in range(NBUF - 1):
            issue(j, j)

        def quad_body(p, carry):
            for b in range(NBUF):
                i = NBUF * p + b
                nxt = i + (NBUF - 1)

                @pl.when(nxt < nchunk)
                def _():
                    issue(nxt, (b + NBUF - 1) % NBUF)

                g_wait(i, b)

                for r in range(C):
                    for c in range(D2 // 16):
                        sl = pl.ds(c * 16, 16)

                        def tsum(lo, hi):
                            if hi - lo == 1:
                                return plsc.bitcast(gbuf[b, r * K + lo, sl],
                                                    jnp.bfloat16)
                            mid = (lo + hi) // 2
                            return tsum(lo, mid) + tsum(mid, hi)

                        obuf[i * C + r, sl] = plsc.bitcast(tsum(0, K),
                                                           jnp.int32)
            return carry

        lax.fori_loop(0, nchunk // NBUF, quad_body, 0)

        row_base = wid * rows_per_w
        pltpu.sync_copy(obuf, agg_hbm.at[pl.ds(row_base, rows_per_w)])
        for h in range(nself):
            pltpu.make_async_copy(feat_hbm.at[sidx_v.at[h]],
                                  sbuf.at[pl.ds(h * sper, sper)],
                                  ssem.at[h]).wait()
        pltpu.sync_copy(sbuf, self_hbm.at[pl.ds(row_base, rows_per_w)])

    return k(feat_q, neigh_idx, node_idx)


def _tc_matmul_relu(self_feats, agg_sum, w_top, w_bot, inv_k):
    B, D = self_feats.shape
    E = w_top.shape[1]
    BLK = 512

    def mm(s_ref, a_ref, wt_ref, wb_ref, o_ref):
        acc = jnp.dot(s_ref[...], wt_ref[...], preferred_element_type=jnp.float32)
        acc = acc + jnp.dot(a_ref[...], wb_ref[...],
                            preferred_element_type=jnp.float32) * inv_k
        o_ref[...] = jnp.maximum(acc, 0.0)

    return pl.pallas_call(
        mm,
        grid=(B // BLK,),
        in_specs=[
            pl.BlockSpec((BLK, D), lambda i: (i, 0)),
            pl.BlockSpec((BLK, D), lambda i: (i, 0)),
            pl.BlockSpec((D, E), lambda i: (0, 0)),
            pl.BlockSpec((D, E), lambda i: (0, 0)),
        ],
        out_specs=pl.BlockSpec((BLK, E), lambda i: (i, 0)),
        out_shape=jax.ShapeDtypeStruct((B, E), jnp.float32),
    )(self_feats, agg_sum, w_top, w_bot)


def kernel(nodes, to_neighs, features, weight):
    B, K = to_neighs.shape
    N, D = features.shape
    D2 = D // 2
    C = 2
    rows_per_w = B // NW
    nchunk = rows_per_w // C
    nself = 2

    feat_bf = features.astype(jnp.bfloat16)
    feat_q = lax.bitcast_convert_type(feat_bf.reshape(N, D2, 2), jnp.int32)
    w_bf = weight.astype(jnp.bfloat16)
    neigh_idx = to_neighs.astype(jnp.int32).reshape(NW, nchunk, C * K)
    node_idx = nodes.astype(jnp.int32).reshape(NW, nself, rows_per_w // nself)

    self_q, agg_q = _sc_gather_sum(
        feat_q, neigh_idx, node_idx, B=B, K=K, D2=D2, C=C
    )
    self_feats = lax.bitcast_convert_type(self_q, jnp.bfloat16).reshape(B, D)
    agg_sum = lax.bitcast_convert_type(agg_q, jnp.bfloat16).reshape(B, D)
    return _tc_matmul_relu(self_feats, agg_sum, w_bf[:D], w_bf[D:], 1.0 / K)


# speedup vs baseline: 2.9413x; 2.7758x over previous
import functools

import jax
import jax.numpy as jnp
from jax import lax
from jax.experimental import pallas as pl
from jax.experimental.pallas import tpu as pltpu
from jax.experimental.pallas import tpu_sc as plsc

NC = 2
NS = 16
NW = NC * NS
NBUF = 4


def _tc_pack(features):
    N, D = features.shape
    D2 = D // 2
    ROWS = 400
    assert N % ROWS == 0

    def body(x_ref, o_ref):
        xb = x_ref[...].astype(jnp.bfloat16)
        lo = lax.bitcast_convert_type(xb[:, :D2], jnp.uint16).astype(jnp.uint32)
        hi = lax.bitcast_convert_type(xb[:, D2:], jnp.uint16).astype(jnp.uint32)
        o_ref[...] = lo | (hi << 16)

    return pl.pallas_call(
        body,
        grid=(N // ROWS,),
        in_specs=[pl.BlockSpec((ROWS, D), lambda i: (i, 0))],
        out_specs=pl.BlockSpec((ROWS, D2), lambda i: (i, 0)),
        out_shape=jax.ShapeDtypeStruct((N, D2), jnp.uint32),
    )(features)


def _sc_gather_sum(feat_q, neigh_idx, node_idx, *, B, K, D2, C):
    rows_per_w = B // NW
    nchunk = rows_per_w // C
    nself = node_idx.shape[1]
    sper = rows_per_w // nself
    assert nchunk % NBUF == 0
    mesh = plsc.VectorSubcoreMesh(
        core_axis_name="c", subcore_axis_name="s", num_cores=NC, num_subcores=NS
    )

    @functools.partial(
        pl.kernel,
        out_type=(
            jax.ShapeDtypeStruct((B, D2), jnp.uint32),
            jax.ShapeDtypeStruct((B, D2), jnp.uint32),
        ),
        mesh=mesh,
        compiler_params=pltpu.CompilerParams(needs_layout_passes=False),
        scratch_types=[
            pltpu.VMEM((nchunk, C * K), jnp.int32),
            pltpu.VMEM((nself, sper), jnp.int32),
            pltpu.VMEM((NBUF, C * K, D2), jnp.uint32),
            pltpu.VMEM((rows_per_w, D2), jnp.uint32),
            pltpu.VMEM((rows_per_w, D2), jnp.uint32),
            pltpu.SemaphoreType.DMA((NBUF,)),
            pltpu.SemaphoreType.DMA((2,)),
        ],
    )
    def k(feat_hbm, nidx_hbm, sidx_hbm, self_hbm, agg_hbm,
          nidx_v, sidx_v, gbuf, sbuf, obuf, gsem, ssem):
        wid = lax.axis_index("s") * NC + lax.axis_index("c")
        pltpu.sync_copy(nidx_hbm.at[wid], nidx_v)
        pltpu.sync_copy(sidx_hbm.at[wid], sidx_v)

        def issue(i, slot):
            pltpu.async_copy(feat_hbm.at[nidx_v.at[i]], gbuf.at[slot],
                             gsem.at[slot])

        def g_wait(i, slot):
            pltpu.make_async_copy(feat_hbm.at[nidx_v.at[i]], gbuf.at[slot],
                                  gsem.at[slot]).wait()

        for h in range(nself):
            pltpu.async_copy(feat_hbm.at[sidx_v.at[h]],
                             sbuf.at[pl.ds(h * sper, sper)], ssem.at[h])
        for j in range(NBUF - 1):
            issue(j, j)

        def quad_body(p, carry):
            for b in range(NBUF):
                i = NBUF * p + b
                nxt = i + (NBUF - 1)

                @pl.when(nxt < nchunk)
                def _():
                    issue(nxt, (b + NBUF - 1) % NBUF)

                g_wait(i, b)

                for r in range(C):
                    for c in range(D2 // 16):
                        sl = pl.ds(c * 16, 16)

                        def tsum(lo, hi):
                            if hi - lo == 1:
                                return plsc.bitcast(gbuf[b, r * K + lo, sl],
                                                    jnp.bfloat16)
                            mid = (lo + hi) // 2
                            return tsum(lo, mid) + tsum(mid, hi)

                        obuf[i * C + r, sl] = plsc.bitcast(tsum(0, K),
                                                           jnp.uint32)
            return carry

        lax.fori_loop(0, nchunk // NBUF, quad_body, 0)

        row_base = wid * rows_per_w
        pltpu.sync_copy(obuf, agg_hbm.at[pl.ds(row_base, rows_per_w)])
        for h in range(nself):
            pltpu.make_async_copy(feat_hbm.at[sidx_v.at[h]],
                                  sbuf.at[pl.ds(h * sper, sper)],
                                  ssem.at[h]).wait()
        pltpu.sync_copy(sbuf, self_hbm.at[pl.ds(row_base, rows_per_w)])

    return k(feat_q, neigh_idx, node_idx)


def _tc_matmul_relu(self_q, agg_q, w_self, w_agg, inv_k):
    B, D2 = self_q.shape
    E = w_self.shape[1]
    BLK = 512

    def unpack(w):
        lo = lax.bitcast_convert_type((w & 0xFFFF).astype(jnp.uint16),
                                      jnp.bfloat16)
        hi = lax.bitcast_convert_type((w >> 16).astype(jnp.uint16),
                                      jnp.bfloat16)
        return lo, hi

    def mm(s_ref, a_ref, wsl_ref, wsh_ref, wal_ref, wah_ref, o_ref):
        s_lo, s_hi = unpack(s_ref[...])
        a_lo, a_hi = unpack(a_ref[...])
        acc = jnp.dot(s_lo, wsl_ref[...], preferred_element_type=jnp.float32)
        acc = acc + jnp.dot(s_hi, wsh_ref[...], preferred_element_type=jnp.float32)
        agg = jnp.dot(a_lo, wal_ref[...], preferred_element_type=jnp.float32)
        agg = agg + jnp.dot(a_hi, wah_ref[...], preferred_element_type=jnp.float32)
        o_ref[...] = jnp.maximum(acc + agg * inv_k, 0.0)

    wspec = pl.BlockSpec((D2, E), lambda i: (0, 0))
    return pl.pallas_call(
        mm,
        grid=(B // BLK,),
        in_specs=[
            pl.BlockSpec((BLK, D2), lambda i: (i, 0)),
            pl.BlockSpec((BLK, D2), lambda i: (i, 0)),
            wspec, wspec, wspec, wspec,
        ],
        out_specs=pl.BlockSpec((BLK, E), lambda i: (i, 0)),
        out_shape=jax.ShapeDtypeStruct((B, E), jnp.float32),
    )(self_q, agg_q, w_self[:D2], w_self[D2:], w_agg[:D2], w_agg[D2:])


def kernel(nodes, to_neighs, features, weight):
    B, K = to_neighs.shape
    N, D = features.shape
    D2 = D // 2
    C = 2
    rows_per_w = B // NW
    nchunk = rows_per_w // C
    nself = 2

    feat_q = _tc_pack(features)
    w_bf = weight.astype(jnp.bfloat16)
    neigh_idx = to_neighs.astype(jnp.int32).reshape(NW, nchunk, C * K)
    node_idx = nodes.astype(jnp.int32).reshape(NW, nself, rows_per_w // nself)

    self_q, agg_q = _sc_gather_sum(
        feat_q, neigh_idx, node_idx, B=B, K=K, D2=D2, C=C
    )
    return _tc_matmul_relu(self_q, agg_q, w_bf[:D], w_bf[D:], 1.0 / K)
